# baseline (speedup 1.0000x reference)
import functools

import jax
import jax.numpy as jnp
from jax.experimental import pallas as pl
from jax.experimental.pallas import tpu as pltpu

H = 128


def _lrelu(x):
    return jnp.where(x > 0, x, 0.01 * x)


def _affine_kernel(x_ref, w_ref, b_ref, o_ref, *, act):
    y = jnp.dot(x_ref[...], w_ref[...], preferred_element_type=jnp.float32)
    y = y + b_ref[...]
    if act:
        y = _lrelu(y)
    o_ref[...] = y


def _affine(x, w, b, act):
    n, din = x.shape
    dout = w.shape[1]
    blk = 4096
    npad = (n + blk - 1) // blk * blk
    xp = jnp.pad(x, ((0, npad - n), (0, 0)))
    out = pl.pallas_call(
        functools.partial(_affine_kernel, act=act),
        grid=(npad // blk,),
        in_specs=[
            pl.BlockSpec((blk, din), lambda i: (i, 0)),
            pl.BlockSpec((din, dout), lambda i: (0, 0)),
            pl.BlockSpec((dout,), lambda i: (0,)),
        ],
        out_specs=pl.BlockSpec((blk, dout), lambda i: (i, 0)),
        out_shape=jax.ShapeDtypeStruct((npad, dout), jnp.float32),
    )(xp, w, b)
    return out[:n]


def _gcn_conv(x, src, dst, W, b, N, act):
    xw = _affine(x, W, jnp.zeros_like(b), act=False)
    loop = jnp.arange(N, dtype=src.dtype)
    s = jnp.concatenate([src, loop])
    d = jnp.concatenate([dst, loop])
    deg = jnp.zeros((N,), xw.dtype).at[d].add(1.0)
    dis = jnp.where(deg > 0, deg ** -0.5, 0.0)
    norm = dis[s] * dis[d]
    msg = xw[s] * norm[:, None]
    out = jnp.zeros((N, W.shape[1]), xw.dtype).at[d].add(msg) + b
    if act:
        out = _lrelu(out)
    return out


def _gcn_block(x, edge_index, Ws, bs):
    N = x.shape[0]
    src, dst = edge_index[0], edge_index[1]
    h = _gcn_conv(x, src, dst, Ws[0], bs[0], N, act=True)
    h = _gcn_conv(h, src, dst, Ws[1], bs[1], N, act=True)
    h = _gcn_conv(h, src, dst, Ws[2], bs[2], N, act=False)
    return h


def kernel(verts, edge_index0, faces0, edge_index1, subdiv_faces0, subdiv_faces1,
           W00, b00, W01, b01, W02, b02, W10, b10, W11, b11, W12, b12):
    x = verts
    off = _gcn_block(x, edge_index0, (W00, W01, W02), (b00, b01, b02))
    v = (x + off)[None]
    edges = edge_index0.T
    edge_mid = v[:, edges].mean(axis=2)
    face_pt = v[:, faces0].mean(axis=2)
    v1 = jnp.concatenate([v, edge_mid, face_pt], axis=1)
    x = v1[0]
    off = _gcn_block(x, edge_index1, (W10, W11, W12), (b10, b11, b12))
    v = (x + off)[None]
    edges = edge_index1.T
    edge_mid = v[:, edges].mean(axis=2)
    face_pt = v[:, subdiv_faces0].mean(axis=2)
    v2 = jnp.concatenate([v, edge_mid, face_pt], axis=1)
    return v2, subdiv_faces1


# speedup vs baseline: 2.3363x; 2.3363x over previous
import functools

import jax
import jax.numpy as jnp
from jax import lax
from jax.experimental import pallas as pl
from jax.experimental.pallas import tpu as pltpu
from jax.experimental.pallas import tpu_sc as plsc

H = 128
NC = 2
NS = 16
L = 16


def _lrelu(x):
    return jnp.where(x > 0, x, 0.01 * x)


def _affine_kernel(x_ref, w_ref, b_ref, o_ref, *, act):
    y = jnp.dot(x_ref[...], w_ref[...], preferred_element_type=jnp.float32)
    y = y + b_ref[...]
    if act:
        y = _lrelu(y)
    o_ref[...] = y


def _affine(x, w, b, act):
    n, din = x.shape
    dout = w.shape[1]
    blk = 4096
    npad = (n + blk - 1) // blk * blk
    xp = jnp.pad(x, ((0, npad - n), (0, 0)))
    out = pl.pallas_call(
        functools.partial(_affine_kernel, act=act),
        grid=(npad // blk,),
        in_specs=[
            pl.BlockSpec((blk, din), lambda i: (i, 0)),
            pl.BlockSpec((din, dout), lambda i: (0, 0)),
            pl.BlockSpec((dout,), lambda i: (0,)),
        ],
        out_specs=pl.BlockSpec((blk, dout), lambda i: (i, 0)),
        out_shape=jax.ShapeDtypeStruct((npad, dout), jnp.float32),
    )(xp, w, b)
    return out[:n]


def _sc_apply128_body(u_h, esrc_h, edst_h, out_h,
                      esrc_v, edst_v, sstg_v, dstg_v, sidx_v, didx_v,
                      gbuf_v, zbuf_v, acc_sh, sem,
                      *, chunk, n_cps, eb, gb, ept, c):
    cid = lax.axis_index("c")
    sid = lax.axis_index("s")
    ebase = sid * ept
    n_eb = ept // eb
    rows_pt = chunk // NS
    zrows = zbuf_v.shape[0]
    lane = lax.iota(jnp.int32, L)

    def zb(i, carry):
        for j in range(c // L):
            zbuf_v[i, pl.ds(j * L, L)] = jnp.zeros((L,), jnp.float32)
        return carry
    lax.fori_loop(0, zrows, zb, 0)

    for k in range(n_cps):
        ck = 2 * k + cid
        lo = ck * chunk
        for i in range(rows_pt // zrows):
            pltpu.sync_copy(
                zbuf_v, acc_sh.at[pl.ds(sid * rows_pt + i * zrows, zrows)])
        plsc.subcore_barrier()

        def drain_full(t, cur0):
            for q in range(gb // L):
                sidx_v[pl.ds(q * L, L)] = sstg_v[pl.ds(t * gb + q * L, L)]
                didx_v[pl.ds(q * L, L)] = dstg_v[pl.ds(t * gb + q * L, L)]
            pltpu.async_copy(u_h.at[sidx_v], gbuf_v, sem).wait()
            pltpu.sync_copy(gbuf_v, acc_sh.at[didx_v], add=True)
            return cur0

        def eb_body(ib, cur):
            pltpu.sync_copy(esrc_h.at[pl.ds(ebase + ib * eb, eb)], esrc_v)
            pltpu.sync_copy(edst_h.at[pl.ds(ebase + ib * eb, eb)], edst_v)

            def scan(j, cur):
                d16 = edst_v[pl.ds(j * L, L)]
                s16 = esrc_v[pl.ds(j * L, L)]
                m = (d16 >= lo) & (d16 < lo + chunk)
                key = jnp.where(m, d16 - lo, jnp.int32(0x7FFFFFFF))
                skey, sval = plsc.sort_key_val(key, s16)
                dstg_v[pl.ds(cur, L)] = skey
                sstg_v[pl.ds(cur, L)] = sval
                return cur + plsc.all_reduce_population_count(m)[0]

            cur = lax.fori_loop(0, eb // L, scan, cur + jnp.int32(0))
            nd = cur // gb
            lax.fori_loop(0, nd, drain_full, 0)
            rem = cur - nd * gb
            base = nd * gb
            svs = [sstg_v[pl.ds(base + q * L, L)] for q in range(gb // L)]
            dvs = [dstg_v[pl.ds(base + q * L, L)] for q in range(gb // L)]
            for q in range(gb // L):
                sstg_v[pl.ds(q * L, L)] = svs[q]
                dstg_v[pl.ds(q * L, L)] = dvs[q]
            return rem

        cur = lax.fori_loop(0, n_eb, eb_body, jnp.int32(0))

        for q in range(gb // L):
            valid = (lane + q * L) < cur
            sidx_v[pl.ds(q * L, L)] = jnp.where(
                valid, sstg_v[pl.ds(q * L, L)], 0)
            didx_v[pl.ds(q * L, L)] = jnp.where(
                valid, dstg_v[pl.ds(q * L, L)], chunk)
        pltpu.async_copy(u_h.at[sidx_v], gbuf_v, sem).wait()
        pltpu.sync_copy(gbuf_v, acc_sh.at[didx_v], add=True)
        plsc.subcore_barrier()

        pltpu.sync_copy(
            acc_sh.at[pl.ds(sid * rows_pt, rows_pt)],
            out_h.at[pl.ds(ck * chunk + sid * rows_pt, rows_pt)])
    return


@functools.lru_cache(maxsize=None)
def _make_sc_apply128(nrows, epad, chunk, n_cps, eb, gb=128, c=H):
    ept = epad // NS
    mesh = plsc.VectorSubcoreMesh(core_axis_name="c", subcore_axis_name="s")
    stg = eb + gb + L
    body = functools.partial(_sc_apply128_body, chunk=chunk, n_cps=n_cps,
                             eb=eb, gb=gb, ept=ept, c=c)
    return pl.kernel(
        body,
        out_type=jax.ShapeDtypeStruct((2 * n_cps * chunk, c), jnp.float32),
        mesh=mesh,
        compiler_params=pltpu.CompilerParams(needs_layout_passes=False),
        scratch_types=[
            pltpu.VMEM((eb,), jnp.int32),
            pltpu.VMEM((eb,), jnp.int32),
            pltpu.VMEM((stg,), jnp.int32),
            pltpu.VMEM((stg,), jnp.int32),
            pltpu.VMEM((gb,), jnp.int32),
            pltpu.VMEM((gb,), jnp.int32),
            pltpu.VMEM((gb, c), jnp.float32),
            pltpu.VMEM((16, c), jnp.float32),
            pltpu.VMEM_SHARED((chunk + L, c), jnp.float32),
            pltpu.SemaphoreType.DMA,
        ],
    )


def _sc_apply128(u, esrc, edst, n, chunk, eb):
    nrows = u.shape[0]
    k = (n + chunk - 1) // chunk
    n_cps = (k + 1) // 2
    assert nrows >= 2 * n_cps * chunk or True
    need = 2 * n_cps * chunk
    if nrows < need:
        u = jnp.pad(u, ((0, need - nrows), (0, 0)))
    e = esrc.shape[0]
    unit = NS * eb
    epad = (e + unit - 1) // unit * unit
    esrc_p = jnp.pad(esrc, (0, epad - e))
    edst_p = jnp.pad(edst, (0, epad - e), constant_values=0x40000000)
    fn = _make_sc_apply128(need, epad, chunk, n_cps, eb)
    out = fn(u, esrc_p, edst_p)
    return out[:n]


def _gcn_block(x, edge_index, Ws, bs, chunk, eb):
    N = x.shape[0]
    src, dst = edge_index[0], edge_index[1]
    deg = jnp.zeros((N,), jnp.float32).at[dst].add(1.0) + 1.0
    dis = deg ** -0.5
    discol = dis[:, None]

    u0 = discol * x
    s0 = jnp.zeros_like(u0).at[dst].add(u0[src])
    y0 = discol * (s0 + u0)
    h1 = _affine(y0, Ws[0], bs[0], act=True)

    u1 = discol * h1
    s1 = _sc_apply128(u1, src, dst, N, chunk, eb)
    y1 = discol * (s1 + u1)
    h2 = _affine(y1, Ws[1], bs[1], act=True)

    g = _affine(h2, Ws[2], jnp.zeros_like(bs[2]), act=False)
    u2 = discol * g
    s2 = jnp.zeros_like(u2).at[dst].add(u2[src])
    off = discol * (s2 + u2) + bs[2]
    return off


def kernel(verts, edge_index0, faces0, edge_index1, subdiv_faces0, subdiv_faces1,
           W00, b00, W01, b01, W02, b02, W10, b10, W11, b11, W12, b12):
    x = verts
    off = _gcn_block(x, edge_index0, (W00, W01, W02), (b00, b01, b02),
                     chunk=11264, eb=2048)
    v = (x + off)[None]
    edges = edge_index0.T
    edge_mid = v[:, edges].mean(axis=2)
    face_pt = v[:, faces0].mean(axis=2)
    v1 = jnp.concatenate([v, edge_mid, face_pt], axis=1)
    x = v1[0]
    off = _gcn_block(x, edge_index1, (W10, W11, W12), (b10, b11, b12),
                     chunk=11264, eb=4096)
    v = (x + off)[None]
    edges = edge_index1.T
    edge_mid = v[:, edges].mean(axis=2)
    face_pt = v[:, subdiv_faces0].mean(axis=2)
    v2 = jnp.concatenate([v, edge_mid, face_pt], axis=1)
    return v2, subdiv_faces1
